# baseline (speedup 1.0000x reference)
import functools

import jax
import jax.numpy as jnp
from jax import lax
from jax.experimental import pallas as pl
from jax.experimental.pallas import tpu as pltpu
from jax.experimental.pallas import tpu_sc as plsc

_U = 5000
_I = 5000
_N = _U + _I
_D = 128
_E = 320000

_NC = 2
_NS = 16
_NW = _NC * _NS
_EW = _E // _NW
_C = 80
_K = _EW // _C
_NRC = _N // _C
_RPT = -(-_NRC // _NS)
_NB = 3


def _scale_rows(msg, wb, j):
    def _group(g, inner):
        w16 = wb[pl.ds(j * _C + g * 16, 16)]
        for e in range(16):
            row = g * 16 + e
            for d in range(_D // 16):
                msg[row, pl.ds(d * 16, 16)] = (
                    msg[row, pl.ds(d * 16, 16)] * w16[e])
        return inner

    lax.fori_loop(0, _C // 16, _group, 0)


def _spmm_body(x_hbm, row_hbm, col_hbm, w_hbm, part_hbm,
               rb, cb, wb, msg, acc, gsem, ssem, rsem, csem):
    c = lax.axis_index("c")
    s = lax.axis_index("s")
    wid = c * _NS + s

    pltpu.async_copy(w_hbm.at[pl.ds(wid * _EW, _EW)], wb, csem[0])

    zeros16 = jnp.zeros((16,), jnp.float32)

    def _zero_row(i, carry):
        for d in range(_D // 16):
            msg[0][i, pl.ds(d * 16, 16)] = zeros16
        return carry

    lax.fori_loop(0, _C, _zero_row, 0)
    for k in range(_RPT):
        rc = s + k * _NS
        off = pl.multiple_of(rc * _C, _C)

        @pl.when(rc < _NRC)
        def _():
            pltpu.async_copy(msg[0], acc.at[pl.ds(off, _C)], gsem[0])
    for k in range(_RPT):
        rc = s + k * _NS
        off = pl.multiple_of(rc * _C, _C)

        @pl.when(rc < _NRC)
        def _():
            pltpu.make_async_copy(msg[0], acc.at[pl.ds(off, _C)],
                                  gsem[0]).wait()
    plsc.subcore_barrier()

    base = wid * _EW
    for t in range(2):
        pltpu.sync_copy(col_hbm.at[pl.ds(base + t * _C, _C)], cb[t])
        pltpu.async_copy(row_hbm.at[pl.ds(base + t * _C, _C)], rb[t], rsem[t])
        pltpu.async_copy(x_hbm.at[cb[t]], msg[t], gsem[t])
    pltpu.async_copy(col_hbm.at[pl.ds(base + 2 * _C, _C)], cb[2], csem[2])
    pltpu.make_async_copy(w_hbm.at[pl.ds(base, _EW)], wb, csem[0]).wait()

    def _step(j, slot):
        nxt = (slot + 2) % _NB

        @pl.when((j > 0) & (j + 2 < _K))
        def _():
            pltpu.make_async_copy(msg[nxt], acc.at[rb[nxt]],
                                  ssem[nxt]).wait()

        @pl.when(j + 2 < _K)
        def _():
            pltpu.async_copy(row_hbm.at[pl.ds(base + (j + 2) * _C, _C)],
                             rb[nxt], rsem[nxt])
            pltpu.make_async_copy(col_hbm.at[pl.ds(base + (j + 2) * _C, _C)],
                                  cb[nxt], csem[nxt]).wait()
            pltpu.async_copy(x_hbm.at[cb[nxt]], msg[nxt], gsem[nxt])

        pltpu.make_async_copy(x_hbm.at[cb[slot]], msg[slot],
                              gsem[slot]).wait()

        @pl.when(j + 3 < _K)
        def _():
            pltpu.async_copy(col_hbm.at[pl.ds(base + (j + 3) * _C, _C)],
                             cb[slot], csem[slot])

        _scale_rows(msg[slot], wb, j)

        pltpu.make_async_copy(row_hbm.at[pl.ds(base + j * _C, _C)], rb[slot],
                              rsem[slot]).wait()
        pltpu.async_copy(msg[slot], acc.at[rb[slot]], ssem[slot],
                         add=True)

    def _triple(jj, carry):
        j0 = jj * _NB
        _step(j0, 0)

        @pl.when(j0 + 1 < _K)
        def _():
            _step(j0 + 1, 1)

        @pl.when(j0 + 2 < _K)
        def _():
            _step(j0 + 2, 2)

        return carry

    lax.fori_loop(0, -(-_K // _NB), _triple, 0)

    for t in range(_NB):
        slot = (_K - 1 - t) % _NB
        pltpu.make_async_copy(msg[slot], acc.at[rb[slot]],
                              ssem[slot]).wait()
    plsc.subcore_barrier()

    for k in range(_RPT):
        rc = s + k * _NS
        off = pl.multiple_of(rc * _C, _C)

        @pl.when(rc < _NRC)
        def _():
            pltpu.async_copy(acc.at[pl.ds(off, _C)],
                             part_hbm.at[c, pl.ds(off, _C)], gsem[0])
    for k in range(_RPT):
        rc = s + k * _NS
        off = pl.multiple_of(rc * _C, _C)

        @pl.when(rc < _NRC)
        def _():
            pltpu.make_async_copy(acc.at[pl.ds(off, _C)],
                                  part_hbm.at[c, pl.ds(off, _C)],
                                  gsem[0]).wait()


_spmm = functools.partial(
    pl.kernel,
    out_type=jax.ShapeDtypeStruct((_NC, _N, _D), jnp.float32),
    mesh=plsc.VectorSubcoreMesh(core_axis_name="c", subcore_axis_name="s"),
    scratch_types=[
        [pltpu.VMEM((_C,), jnp.int32) for _ in range(_NB)],
        [pltpu.VMEM((_C,), jnp.int32) for _ in range(_NB)],
        pltpu.VMEM((_EW,), jnp.float32),
        [pltpu.VMEM((_C, _D), jnp.float32) for _ in range(_NB)],
        pltpu.VMEM_SHARED((_N, _D), jnp.float32),
        [pltpu.SemaphoreType.DMA for _ in range(_NB)],
        [pltpu.SemaphoreType.DMA for _ in range(_NB)],
        [pltpu.SemaphoreType.DMA for _ in range(_NB)],
        [pltpu.SemaphoreType.DMA for _ in range(_NB)],
    ],
)(_spmm_body)


_CB = 1000


def _concat_body(u_ref, i_ref, o_ref):
    o_ref[0] = u_ref[...]
    o_ref[1] = i_ref[...]


def _concat(user_emb, item_emb):
    nb = _U // _CB
    spec = pl.BlockSpec((_CB, _D), lambda j: (j, 0))
    out = pl.pallas_call(
        _concat_body,
        grid=(nb,),
        in_specs=[spec, spec],
        out_specs=pl.BlockSpec((2, _CB, _D), lambda j: (0, j, 0)),
        out_shape=jax.ShapeDtypeStruct((2, _U, _D), jnp.float32),
    )(user_emb, item_emb)
    return out.reshape(_N, _D)


_RB = 1000


def _reduce_body(pu0, pu1, pi0, pi1, u_ref, i_ref):
    u_ref[...] = pu0[0] + pu1[0]
    i_ref[...] = pi0[0] + pi1[0]


def _reduce(partial):
    nb = _U // _RB
    spec = lambda co, ro: pl.BlockSpec((1, _RB, _D), lambda j: (co, j + ro, 0))
    out_spec = pl.BlockSpec((_RB, _D), lambda j: (j, 0))
    return pl.pallas_call(
        _reduce_body,
        grid=(nb,),
        in_specs=[spec(0, 0), spec(1, 0), spec(0, nb), spec(1, nb)],
        out_specs=[out_spec, out_spec],
        out_shape=[jax.ShapeDtypeStruct((_U, _D), jnp.float32),
                   jax.ShapeDtypeStruct((_I, _D), jnp.float32)],
    )(partial, partial, partial, partial)


def kernel(user_emb, item_emb, edge_index, edge_weight):
    x = _concat(user_emb, item_emb)
    partial = _spmm(x, edge_index[0], edge_index[1], edge_weight)
    return _reduce(partial)


# speedup vs baseline: 1.0113x; 1.0113x over previous
import functools

import jax
import jax.numpy as jnp
from jax import lax
from jax.experimental import pallas as pl
from jax.experimental.pallas import tpu as pltpu
from jax.experimental.pallas import tpu_sc as plsc

_U = 5000
_I = 5000
_N = _U + _I
_D = 128
_E = 320000

_NC = 2
_NS = 16
_NW = _NC * _NS
_EW = _E // _NW
_C = 80
_K = _EW // _C
_NRC = _N // _C
_RPT = -(-_NRC // _NS)
_NB = 3


def _scale_rows(msg, wb, j):
    def _group(g, inner):
        w16 = wb[pl.ds(j * _C + g * 16, 16)]
        for e in range(16):
            row = g * 16 + e
            for d in range(_D // 16):
                msg[row, pl.ds(d * 16, 16)] = (
                    msg[row, pl.ds(d * 16, 16)] * w16[e])
        return inner

    lax.fori_loop(0, _C // 16, _group, 0)


def _spmm_body(x_hbm, row_hbm, col_hbm, w_hbm, part_hbm,
               rb, cb, wb, msg, acc, gsem, ssem, rsem, csem):
    c = lax.axis_index("c")
    s = lax.axis_index("s")
    wid = c * _NS + s

    pltpu.async_copy(w_hbm.at[pl.ds(wid * _EW, _EW)], wb, csem[0])

    zeros16 = jnp.zeros((16,), jnp.float32)

    def _zero_row(i, carry):
        for d in range(_D // 16):
            msg[0][i, pl.ds(d * 16, 16)] = zeros16
        return carry

    lax.fori_loop(0, _C, _zero_row, 0)
    for k in range(_RPT):
        rc = s + k * _NS
        off = pl.multiple_of(rc * _C, _C)

        @pl.when(rc < _NRC)
        def _():
            pltpu.async_copy(msg[0], acc.at[pl.ds(off, _C)], gsem[0])
    for k in range(_RPT):
        rc = s + k * _NS
        off = pl.multiple_of(rc * _C, _C)

        @pl.when(rc < _NRC)
        def _():
            pltpu.make_async_copy(msg[0], acc.at[pl.ds(off, _C)],
                                  gsem[0]).wait()
    plsc.subcore_barrier()

    base = wid * _EW
    for t in range(2):
        pltpu.sync_copy(col_hbm.at[pl.ds(base + t * _C, _C)], cb[t])
        pltpu.async_copy(row_hbm.at[pl.ds(base + t * _C, _C)], rb[t], rsem[t])
        pltpu.async_copy(x_hbm.at[cb[t]], msg[t], gsem[t])
    pltpu.async_copy(col_hbm.at[pl.ds(base + 2 * _C, _C)], cb[2], csem[2])
    pltpu.make_async_copy(w_hbm.at[pl.ds(base, _EW)], wb, csem[0]).wait()

    def _step(j, slot):
        nxt = (slot + 2) % _NB

        @pl.when((j > 0) & (j + 2 < _K))
        def _():
            pltpu.make_async_copy(msg[nxt], acc.at[rb[nxt]],
                                  ssem[nxt]).wait()

        @pl.when(j + 2 < _K)
        def _():
            pltpu.async_copy(row_hbm.at[pl.ds(base + (j + 2) * _C, _C)],
                             rb[nxt], rsem[nxt])
            pltpu.make_async_copy(col_hbm.at[pl.ds(base + (j + 2) * _C, _C)],
                                  cb[nxt], csem[nxt]).wait()
            pltpu.async_copy(x_hbm.at[cb[nxt]], msg[nxt], gsem[nxt])

        pltpu.make_async_copy(x_hbm.at[cb[slot]], msg[slot],
                              gsem[slot]).wait()

        @pl.when(j + 3 < _K)
        def _():
            pltpu.async_copy(col_hbm.at[pl.ds(base + (j + 3) * _C, _C)],
                             cb[slot], csem[slot])

        _scale_rows(msg[slot], wb, j)

        pltpu.make_async_copy(row_hbm.at[pl.ds(base + j * _C, _C)], rb[slot],
                              rsem[slot]).wait()
        pltpu.async_copy(msg[slot], acc.at[rb[slot]], ssem[slot],
                         add=True)

    def _triple(jj, carry):
        j0 = jj * _NB
        _step(j0, 0)

        @pl.when(j0 + 1 < _K)
        def _():
            _step(j0 + 1, 1)

        @pl.when(j0 + 2 < _K)
        def _():
            _step(j0 + 2, 2)

        return carry

    lax.fori_loop(0, -(-_K // _NB), _triple, 0)

    for t in range(_NB):
        slot = (_K - 1 - t) % _NB
        pltpu.make_async_copy(msg[slot], acc.at[rb[slot]],
                              ssem[slot]).wait()
    plsc.subcore_barrier()

    for k in range(_RPT):
        rc = s + k * _NS
        off = pl.multiple_of(rc * _C, _C)

        @pl.when(rc < _NRC)
        def _():
            pltpu.async_copy(acc.at[pl.ds(off, _C)],
                             part_hbm.at[c, pl.ds(off, _C)], gsem[0])
    for k in range(_RPT):
        rc = s + k * _NS
        off = pl.multiple_of(rc * _C, _C)

        @pl.when(rc < _NRC)
        def _():
            pltpu.make_async_copy(acc.at[pl.ds(off, _C)],
                                  part_hbm.at[c, pl.ds(off, _C)],
                                  gsem[0]).wait()


_spmm = functools.partial(
    pl.kernel,
    out_type=jax.ShapeDtypeStruct((_NC, _N, _D), jnp.float32),
    mesh=plsc.VectorSubcoreMesh(core_axis_name="c", subcore_axis_name="s"),
    scratch_types=[
        [pltpu.VMEM((_C,), jnp.int32) for _ in range(_NB)],
        [pltpu.VMEM((_C,), jnp.int32) for _ in range(_NB)],
        pltpu.VMEM((_EW,), jnp.float32),
        [pltpu.VMEM((_C, _D), jnp.float32) for _ in range(_NB)],
        pltpu.VMEM_SHARED((_N, _D), jnp.float32),
        [pltpu.SemaphoreType.DMA for _ in range(_NB)],
        [pltpu.SemaphoreType.DMA for _ in range(_NB)],
        [pltpu.SemaphoreType.DMA for _ in range(_NB)],
        [pltpu.SemaphoreType.DMA for _ in range(_NB)],
    ],
)(_spmm_body)


_CB = 1000


def _concat_body(u_ref, i_ref, o_ref):
    o_ref[0] = u_ref[...]
    o_ref[1] = i_ref[...]


def _concat(user_emb, item_emb):
    nb = _U // _CB
    spec = pl.BlockSpec((_CB, _D), lambda j: (j, 0))
    out = pl.pallas_call(
        _concat_body,
        grid=(nb,),
        in_specs=[spec, spec],
        out_specs=pl.BlockSpec((2, _CB, _D), lambda j: (0, j, 0)),
        out_shape=jax.ShapeDtypeStruct((2, _U, _D), jnp.float32),
    )(user_emb, item_emb)
    return out.reshape(_N, _D)


_RB = 1000


def _reduce_body(pu0, pu1, pi0, pi1, u_ref, i_ref):
    u_ref[...] = pu0[0] + pu1[0]
    i_ref[...] = pi0[0] + pi1[0]


def _reduce(partial):
    nb = _U // _RB
    spec = lambda co, ro: pl.BlockSpec((1, _RB, _D), lambda j: (co, j + ro, 0))
    out_spec = pl.BlockSpec((_RB, _D), lambda j: (j, 0))
    return pl.pallas_call(
        _reduce_body,
        grid=(nb,),
        in_specs=[spec(0, 0), spec(1, 0), spec(0, nb), spec(1, nb)],
        out_specs=[out_spec, out_spec],
        out_shape=[jax.ShapeDtypeStruct((_U, _D), jnp.float32),
                   jax.ShapeDtypeStruct((_I, _D), jnp.float32)],
    )(partial, partial, partial, partial)


def kernel(user_emb, item_emb, edge_index, edge_weight):
    x = jnp.concatenate([user_emb, item_emb], axis=0)
    partial = _spmm(x, edge_index[0], edge_index[1], edge_weight)
    return _reduce(partial)


# speedup vs baseline: 1.0788x; 1.0667x over previous
import functools

import jax
import jax.numpy as jnp
from jax import lax
from jax.experimental import pallas as pl
from jax.experimental.pallas import tpu as pltpu
from jax.experimental.pallas import tpu_sc as plsc

_U = 5000
_I = 5000
_N = _U + _I
_D = 128
_E = 320000

_NC = 2
_NS = 16
_NW = _NC * _NS
_EW = _E // _NW
_C = 80
_K = _EW // _C
_NRC = _N // _C
_RPT = -(-_NRC // _NS)
_NB = 3


def _scale_rows(msg, wb, j):
    def _group(g, inner):
        w16 = wb[pl.ds(j * _C + g * 16, 16)]
        for e in range(16):
            row = g * 16 + e
            for d in range(_D // 16):
                msg[row, pl.ds(d * 16, 16)] = (
                    msg[row, pl.ds(d * 16, 16)] * w16[e])
        return inner

    lax.fori_loop(0, _C // 16, _group, 0)


def _spmm_body(x_hbm, pk_hbm, w_hbm, part_hbm,
               rb, cb, wb, msg, acc, gsem, ssem, rsem, csem):
    c = lax.axis_index("c")
    s = lax.axis_index("s")
    wid = c * _NS + s

    pltpu.async_copy(w_hbm.at[pl.ds(wid * _EW, _EW)], wb, csem[0])

    zeros16 = jnp.zeros((16,), jnp.float32)

    def _zero_row(i, carry):
        for d in range(_D // 16):
            msg[0][i, pl.ds(d * 16, 16)] = zeros16
        return carry

    lax.fori_loop(0, _C, _zero_row, 0)
    for k in range(_RPT):
        rc = s + k * _NS
        off = pl.multiple_of(rc * _C, _C)

        @pl.when(rc < _NRC)
        def _():
            pltpu.async_copy(msg[0], acc.at[pl.ds(off, _C)], gsem[0])
    for k in range(_RPT):
        rc = s + k * _NS
        off = pl.multiple_of(rc * _C, _C)

        @pl.when(rc < _NRC)
        def _():
            pltpu.make_async_copy(msg[0], acc.at[pl.ds(off, _C)],
                                  gsem[0]).wait()
    plsc.subcore_barrier()

    base = wid * _EW
    for t in range(2):
        pltpu.sync_copy(pk_hbm.at[pl.ds(_E + base + t * _C, _C)], cb[t])
        pltpu.async_copy(pk_hbm.at[pl.ds(base + t * _C, _C)], rb[t], rsem[t])
        pltpu.async_copy(x_hbm.at[cb[t]], msg[t], gsem[t])
    pltpu.async_copy(pk_hbm.at[pl.ds(_E + base + 2 * _C, _C)], cb[2], csem[2])
    pltpu.make_async_copy(w_hbm.at[pl.ds(base, _EW)], wb, csem[0]).wait()

    def _step(j, slot):
        nxt = (slot + 2) % _NB

        @pl.when((j > 0) & (j + 2 < _K))
        def _():
            pltpu.make_async_copy(msg[nxt], acc.at[rb[nxt]],
                                  ssem[nxt]).wait()

        @pl.when(j + 2 < _K)
        def _():
            pltpu.async_copy(pk_hbm.at[pl.ds(base + (j + 2) * _C, _C)],
                             rb[nxt], rsem[nxt])
            pltpu.make_async_copy(pk_hbm.at[pl.ds(_E + base + (j + 2) * _C, _C)],
                                  cb[nxt], csem[nxt]).wait()
            pltpu.async_copy(x_hbm.at[cb[nxt]], msg[nxt], gsem[nxt])

        pltpu.make_async_copy(x_hbm.at[cb[slot]], msg[slot],
                              gsem[slot]).wait()

        @pl.when(j + 3 < _K)
        def _():
            pltpu.async_copy(pk_hbm.at[pl.ds(_E + base + (j + 3) * _C, _C)],
                             cb[slot], csem[slot])

        _scale_rows(msg[slot], wb, j)

        pltpu.make_async_copy(pk_hbm.at[pl.ds(base + j * _C, _C)], rb[slot],
                              rsem[slot]).wait()
        pltpu.async_copy(msg[slot], acc.at[rb[slot]], ssem[slot],
                         add=True)

    def _triple(jj, carry):
        j0 = jj * _NB
        _step(j0, 0)

        @pl.when(j0 + 1 < _K)
        def _():
            _step(j0 + 1, 1)

        @pl.when(j0 + 2 < _K)
        def _():
            _step(j0 + 2, 2)

        return carry

    lax.fori_loop(0, -(-_K // _NB), _triple, 0)

    for t in range(_NB):
        slot = (_K - 1 - t) % _NB
        pltpu.make_async_copy(msg[slot], acc.at[rb[slot]],
                              ssem[slot]).wait()
    plsc.subcore_barrier()

    for k in range(_RPT):
        rc = s + k * _NS
        off = pl.multiple_of(rc * _C, _C)

        @pl.when(rc < _NRC)
        def _():
            pltpu.async_copy(acc.at[pl.ds(off, _C)],
                             part_hbm.at[c, pl.ds(off, _C)], gsem[0])
    for k in range(_RPT):
        rc = s + k * _NS
        off = pl.multiple_of(rc * _C, _C)

        @pl.when(rc < _NRC)
        def _():
            pltpu.make_async_copy(acc.at[pl.ds(off, _C)],
                                  part_hbm.at[c, pl.ds(off, _C)],
                                  gsem[0]).wait()


_spmm = functools.partial(
    pl.kernel,
    out_type=jax.ShapeDtypeStruct((_NC, _N, _D), jnp.float32),
    mesh=plsc.VectorSubcoreMesh(core_axis_name="c", subcore_axis_name="s"),
    scratch_types=[
        [pltpu.VMEM((_C,), jnp.int32) for _ in range(_NB)],
        [pltpu.VMEM((_C,), jnp.int32) for _ in range(_NB)],
        pltpu.VMEM((_EW,), jnp.float32),
        [pltpu.VMEM((_C, _D), jnp.float32) for _ in range(_NB)],
        pltpu.VMEM_SHARED((_N, _D), jnp.float32),
        [pltpu.SemaphoreType.DMA for _ in range(_NB)],
        [pltpu.SemaphoreType.DMA for _ in range(_NB)],
        [pltpu.SemaphoreType.DMA for _ in range(_NB)],
        [pltpu.SemaphoreType.DMA for _ in range(_NB)],
    ],
)(_spmm_body)


_RB = 1000


def _reduce_body(pu0, pu1, pi0, pi1, u_ref, i_ref):
    u_ref[...] = pu0[0] + pu1[0]
    i_ref[...] = pi0[0] + pi1[0]


def _reduce(partial):
    nb = _U // _RB
    spec = lambda co, ro: pl.BlockSpec((1, _RB, _D), lambda j: (co, j + ro, 0))
    out_spec = pl.BlockSpec((_RB, _D), lambda j: (j, 0))
    return pl.pallas_call(
        _reduce_body,
        grid=(nb,),
        in_specs=[spec(0, 0), spec(1, 0), spec(0, nb), spec(1, nb)],
        out_specs=[out_spec, out_spec],
        out_shape=[jax.ShapeDtypeStruct((_U, _D), jnp.float32),
                   jax.ShapeDtypeStruct((_I, _D), jnp.float32)],
    )(partial, partial, partial, partial)


def kernel(user_emb, item_emb, edge_index, edge_weight):
    x = jnp.concatenate([user_emb, item_emb], axis=0)
    pk = edge_index.reshape(2 * _E)
    partial = _spmm(x, pk, edge_weight)
    return _reduce(partial)
